# baseline (speedup 1.0000x reference)
import jax
import jax.numpy as jnp
from jax.experimental import pallas as pl
from jax.experimental.pallas import tpu as pltpu
from jax.experimental.pallas import tpu_sc as plsc

EMBED_DIM = 64
RANK_DIM = 8
SCALING = 16.0 / 8.0
GATHER_WINDOW = 128
PACK_ROWS = 2000
OUT_BATCH = 64


def _tc_w_ts(a_t, b_t):
    def body(at_ref, bt_ref, w_ref):
        w_ref[...] = SCALING * jnp.dot(at_ref[...], bt_ref[...],
                                       preferred_element_type=jnp.float32)

    return pl.pallas_call(
        body,
        out_shape=jax.ShapeDtypeStruct((EMBED_DIM, EMBED_DIM), jnp.float32),
    )(a_t, b_t)


def _tc_pack_bf16(table):
    quarter = table.shape[0] // 4
    n_blocks = quarter // PACK_ROWS

    def pack2(t):
        b = t.astype(jnp.bfloat16)
        u = jax.lax.bitcast_convert_type(b, jnp.uint16)
        lo = u[:, :32].astype(jnp.uint32)
        hi = u[:, 32:].astype(jnp.uint32)
        return jax.lax.bitcast_convert_type(lo | (hi << 16), jnp.float32)

    def body(q0_ref, q1_ref, q2_ref, q3_ref, o_ref):
        o_ref[...] = jnp.concatenate(
            [pack2(q0_ref[...]), pack2(q1_ref[...]),
             pack2(q2_ref[...]), pack2(q3_ref[...])], axis=1)

    in_specs = [
        pl.BlockSpec((PACK_ROWS, EMBED_DIM),
                     lambda i, k=k: (i + k * n_blocks, 0))
        for k in range(4)
    ]
    return pl.pallas_call(
        body,
        grid=(n_blocks,),
        in_specs=in_specs,
        out_specs=pl.BlockSpec((PACK_ROWS, 2 * EMBED_DIM), lambda i: (i, 0)),
        out_shape=jax.ShapeDtypeStruct((quarter, 2 * EMBED_DIM), jnp.float32),
    )(table, table, table, table)


def _sc_gather(table_wide, idx_q):
    n = idx_q.shape[0]
    width = table_wide.shape[1]
    indices = idx_q.reshape(1, n)
    mesh = plsc.VectorSubcoreMesh(core_axis_name="core",
                                  subcore_axis_name="subcore")

    @pl.kernel(out_type=jax.ShapeDtypeStruct((n, width), table_wide.dtype),
               mesh=mesh)
    def gather_kernel(tab_hbm, i_hbm, o_hbm):
        def body(i_vmem, o_vmem):
            pltpu.sync_copy(tab_hbm.at[i_vmem.at[0]], o_vmem)

        pltpu.emit_pipeline(
            body,
            grid=(n // GATHER_WINDOW,),
            in_specs=[pl.BlockSpec((1, GATHER_WINDOW), lambda i: (0, i))],
            out_specs=[pl.BlockSpec((GATHER_WINDOW, width),
                                    lambda i: (i, 0))],
            core_axis_name=("core", "subcore"),
            dimension_semantics=(pltpu.PARALLEL,),
        )(i_hbm, o_hbm)

    return gather_kernel(table_wide, indices)


def _tc_select_lora(g_wide, b1_t, b0_t, w_ts, bsz, seq):
    rows_per_block = OUT_BATCH * seq
    par_cols = rows_per_block // 128

    def body(g_ref, b1_ref, b0_ref, w_ref, o_ref):
        gu = jax.lax.bitcast_convert_type(g_ref[...], jnp.uint32)
        parts = []
        for j in range(par_cols):
            lo, hi = j * 128, (j + 1) * 128
            m1 = b1_ref[0, :, j:j + 1] > 0.5
            m0 = b0_ref[0, :, j:j + 1] > 0.5
            h = jnp.where(m1, gu[lo:hi, EMBED_DIM:], gu[lo:hi, :EMBED_DIM])
            q = jnp.where(m0, h[:, 32:], h[:, :32])
            lo16 = jax.lax.bitcast_convert_type(q << 16, jnp.float32)
            hi16 = jax.lax.bitcast_convert_type(
                q & jnp.uint32(0xFFFF0000), jnp.float32)
            parts.append(jnp.concatenate([lo16, hi16], axis=1))
        sel = jnp.concatenate(parts, axis=0)
        out = sel + jnp.dot(sel, w_ref[...],
                            preferred_element_type=jnp.float32)
        o_ref[...] = out.reshape(OUT_BATCH, seq, EMBED_DIM)

    return pl.pallas_call(
        body,
        grid=(bsz // OUT_BATCH,),
        in_specs=[
            pl.BlockSpec((rows_per_block, 2 * EMBED_DIM), lambda i: (i, 0)),
            pl.BlockSpec((1, 128, par_cols), lambda i: (i, 0, 0)),
            pl.BlockSpec((1, 128, par_cols), lambda i: (i, 0, 0)),
            pl.BlockSpec((EMBED_DIM, EMBED_DIM), lambda i: (0, 0)),
        ],
        out_specs=pl.BlockSpec((OUT_BATCH, seq, EMBED_DIM),
                               lambda i: (i, 0, 0)),
        out_shape=jax.ShapeDtypeStruct((bsz, seq, EMBED_DIM), jnp.float32),
    )(g_wide, b1_t, b0_t, w_ts)


def _bit_cols(bits_f32, n, bsz, par_cols):
    return (bits_f32
            .reshape(n // 128, 128).T
            .reshape(128, bsz // OUT_BATCH, par_cols)
            .transpose(1, 0, 2))


def kernel(x, table, lora_A, lora_B):
    bsz, seq = x.shape
    n = bsz * seq
    par_cols = OUT_BATCH * seq // 128
    quarter = table.shape[0] // 4
    idx = x.reshape(-1).astype(jnp.int32)
    qsel = idx // quarter
    b1_t = _bit_cols((qsel >> 1).astype(jnp.float32), n, bsz, par_cols)
    b0_t = _bit_cols((qsel & 1).astype(jnp.float32), n, bsz, par_cols)
    w_ts = _tc_w_ts(lora_A.T, lora_B.T)
    table_wide = _tc_pack_bf16(table)
    g_wide = _sc_gather(table_wide, idx % quarter)
    return _tc_select_lora(g_wide, b1_t, b0_t, w_ts, bsz, seq)
